# Initial kernel scaffold; baseline (speedup 1.0000x reference)
import functools

import jax
import jax.numpy as jnp
from jax import lax
from jax.experimental import pallas as pl
from jax.experimental.pallas import tpu as pltpu
from jax.experimental.pallas import tpu_sc as plsc

_NC = 2
_NS = 16
_K = 80


def _sc_degree(n, e):
    nw = _NC * _NS
    ew = e // nw
    assert e % nw == 0 and ew % _K == 0 and n % _NS == 0
    nchunks = ew // _K
    rpt = n // _NS
    nz, rem = rpt // _K, rpt % _K
    mesh = plsc.VectorSubcoreMesh(core_axis_name="c", subcore_axis_name="s")

    @functools.partial(
        pl.kernel,
        out_type=jax.ShapeDtypeStruct((_NC, n, 16), jnp.float32),
        mesh=mesh,
        scratch_types=[
            pltpu.VMEM((_K,), jnp.int32),
            pltpu.VMEM((_K,), jnp.float32),
            pltpu.VMEM((_K, 16), jnp.float32),
            pltpu.VMEM_SHARED((n, 16), jnp.float32),
        ],
    )
    def kern(dst_hbm, w_hbm, out_hbm, dstv, wv, rows, deg_sh):
        c = lax.axis_index("c")
        s = lax.axis_index("s")
        wid = c * _NS + s
        zvec = jnp.zeros((16,), jnp.float32)

        @pl.loop(0, _K)
        def _zero_rows(i):
            rows[i, :] = zvec

        row0 = s * rpt
        for k in range(nz):
            pltpu.sync_copy(rows.at[pl.ds(0, _K)], deg_sh.at[pl.ds(row0 + k * _K, _K)])
        if rem:
            pltpu.sync_copy(rows.at[pl.ds(0, rem)], deg_sh.at[pl.ds(row0 + nz * _K, rem)])
        plsc.subcore_barrier()

        col_iota = lax.iota(jnp.int32, 16)
        zero_col = jnp.zeros((16,), jnp.int32)

        @pl.loop(0, nchunks)
        def _chunk(j):
            base = wid * ew + j * _K
            pltpu.sync_copy(dst_hbm.at[pl.ds(base, _K)], dstv)
            pltpu.sync_copy(w_hbm.at[pl.ds(base, _K)], wv)
            for g in range(_K // 16):
                wvec = wv[pl.ds(g * 16, 16)]
                plsc.store_scatter(rows, [col_iota + g * 16, zero_col], wvec)
            pltpu.sync_copy(rows, deg_sh.at[dstv], add=True)

        plsc.subcore_barrier()
        pltpu.sync_copy(deg_sh.at[pl.ds(row0, rpt)], out_hbm.at[c].at[pl.ds(row0, rpt)])

    return kern


def _sc_scatter(n, e, f):
    nw = _NC * _NS
    ew = e // nw
    assert e % nw == 0 and ew % _K == 0 and n % _NS == 0 and f % 16 == 0
    nchunks = ew // _K
    rpt = n // _NS
    nz, rem = rpt // _K, rpt % _K
    mesh = plsc.VectorSubcoreMesh(core_axis_name="c", subcore_axis_name="s")

    @functools.partial(
        pl.kernel,
        out_type=jax.ShapeDtypeStruct((_NC, n, f), jnp.float32),
        mesh=mesh,
        scratch_types=[
            pltpu.VMEM((_K,), jnp.int32),
            pltpu.VMEM((_K,), jnp.int32),
            pltpu.VMEM((_K,), jnp.float32),
            pltpu.VMEM((_K, f), jnp.float32),
            pltpu.VMEM_SHARED((n, f), jnp.float32),
            pltpu.SemaphoreType.DMA,
        ],
    )
    def kern(y_hbm, src_hbm, dst_hbm, w_hbm, out_hbm, srcv, dstv, wv, rows, acc, sem):
        c = lax.axis_index("c")
        s = lax.axis_index("s")
        wid = c * _NS + s
        zvec = jnp.zeros((16,), jnp.float32)

        @pl.loop(0, _K)
        def _zero_rows(i):
            for g in range(f // 16):
                rows[i, pl.ds(g * 16, 16)] = zvec

        row0 = s * rpt
        for k in range(nz):
            pltpu.sync_copy(rows.at[pl.ds(0, _K)], acc.at[pl.ds(row0 + k * _K, _K)])
        if rem:
            pltpu.sync_copy(rows.at[pl.ds(0, rem)], acc.at[pl.ds(row0 + nz * _K, rem)])
        plsc.subcore_barrier()

        @pl.loop(0, nchunks)
        def _chunk(j):
            base = wid * ew + j * _K
            pltpu.sync_copy(src_hbm.at[pl.ds(base, _K)], srcv)
            pltpu.sync_copy(dst_hbm.at[pl.ds(base, _K)], dstv)
            pltpu.sync_copy(w_hbm.at[pl.ds(base, _K)], wv)
            pltpu.async_copy(y_hbm.at[srcv], rows, sem).wait()

            @pl.loop(0, _K)
            def _scale(i):
                wi = wv[i]
                for g in range(f // 16):
                    rows[i, pl.ds(g * 16, 16)] = rows[i, pl.ds(g * 16, 16)] * wi

            pltpu.sync_copy(rows, acc.at[dstv], add=True)

        plsc.subcore_barrier()
        pltpu.sync_copy(acc.at[pl.ds(row0, rpt)], out_hbm.at[c].at[pl.ds(row0, rpt)])

    return kern


def _tc_mm1(n, fin, fh, blk):
    def body(deg_ref, x_ref, w_ref, y_ref, dinv_ref):
        d = deg_ref[0, :, 0:1] + deg_ref[1, :, 0:1] + 1.0
        dinv = jnp.where(d > 0, lax.rsqrt(d), 0.0)
        xw = jnp.dot(x_ref[...], w_ref[...], preferred_element_type=jnp.float32,
                     precision=lax.Precision.HIGHEST)
        y_ref[...] = dinv * xw
        dinv_ref[...] = dinv

    return pl.pallas_call(
        body,
        grid=(n // blk,),
        in_specs=[
            pl.BlockSpec((_NC, blk, 16), lambda i: (0, i, 0)),
            pl.BlockSpec((blk, fin), lambda i: (i, 0)),
            pl.BlockSpec((fin, fh), lambda i: (0, 0)),
        ],
        out_specs=[
            pl.BlockSpec((blk, fh), lambda i: (i, 0)),
            pl.BlockSpec((blk, 1), lambda i: (i, 0)),
        ],
        out_shape=[
            jax.ShapeDtypeStruct((n, fh), jnp.float32),
            jax.ShapeDtypeStruct((n, 1), jnp.float32),
        ],
    )


def _tc_mm2(n, fh, fo, blk):
    def body(acc_ref, y1_ref, dinv_ref, b1_ref, w2_ref, y2_ref):
        dinv = dinv_ref[...]
        a = acc_ref[0] + acc_ref[1] + y1_ref[...]
        h = jnp.maximum(dinv * a + b1_ref[...], 0.0)
        y2_ref[...] = dinv * jnp.dot(h, w2_ref[...], preferred_element_type=jnp.float32,
                                     precision=lax.Precision.HIGHEST)

    return pl.pallas_call(
        body,
        grid=(n // blk,),
        in_specs=[
            pl.BlockSpec((_NC, blk, fh), lambda i: (0, i, 0)),
            pl.BlockSpec((blk, fh), lambda i: (i, 0)),
            pl.BlockSpec((blk, 1), lambda i: (i, 0)),
            pl.BlockSpec((1, fh), lambda i: (0, 0)),
            pl.BlockSpec((fh, fo), lambda i: (0, 0)),
        ],
        out_specs=pl.BlockSpec((blk, fo), lambda i: (i, 0)),
        out_shape=jax.ShapeDtypeStruct((n, fo), jnp.float32),
    )


def _tc_fin(n, fo, blk):
    def body(acc_ref, y2_ref, dinv_ref, b2_ref, out_ref):
        out_ref[...] = dinv_ref[...] * (acc_ref[0] + acc_ref[1] + y2_ref[...]) + b2_ref[...]

    return pl.pallas_call(
        body,
        grid=(n // blk,),
        in_specs=[
            pl.BlockSpec((_NC, blk, fo), lambda i: (0, i, 0)),
            pl.BlockSpec((blk, fo), lambda i: (i, 0)),
            pl.BlockSpec((blk, 1), lambda i: (i, 0)),
            pl.BlockSpec((1, fo), lambda i: (0, 0)),
        ],
        out_specs=pl.BlockSpec((blk, fo), lambda i: (i, 0)),
        out_shape=jax.ShapeDtypeStruct((n, fo), jnp.float32),
    )


def kernel(in_feat, edge_index, edge_weight, W1, b1, W2, b2):
    n, fin = in_feat.shape
    e = edge_index.shape[1]
    fh = W1.shape[1]
    fo = W2.shape[1]
    blk = 1000
    assert n % blk == 0

    src = edge_index[0].astype(jnp.int32)
    dst = edge_index[1].astype(jnp.int32)
    w = edge_weight.astype(jnp.float32)

    deg2 = _sc_degree(n, e)(dst, w)
    y1, dinv = _tc_mm1(n, fin, fh, blk)(deg2, in_feat.astype(jnp.float32), W1)
    acc1 = _sc_scatter(n, e, fh)(y1, src, dst, w)
    y2 = _tc_mm2(n, fh, fo, blk)(acc1, y1, dinv, b1.reshape(1, fh), W2)
    acc2 = _sc_scatter(n, e, fo)(y2, src, dst, w)
    out = _tc_fin(n, fo, blk)(acc2, y2, dinv, b2.reshape(1, fo))
    return out


# speedup vs baseline: 9.0168x; 9.0168x over previous
import functools

import jax
import jax.numpy as jnp
from jax import lax
from jax.experimental import pallas as pl
from jax.experimental.pallas import tpu as pltpu
from jax.experimental.pallas import tpu_sc as plsc

_NC = 2
_NS = 16
_K = 80


def _sc_degree(n, e):
    nw = _NC * _NS
    ew = e // nw
    assert e % nw == 0 and ew % _K == 0 and n % _NS == 0
    nchunks = ew // _K
    rpt = (n // 8 // _NS) * 8
    tail = n - rpt * _NS
    nz, rem = rpt // _K, rpt % _K
    mesh = plsc.VectorSubcoreMesh(core_axis_name="c", subcore_axis_name="s")

    @functools.partial(
        pl.kernel,
        out_type=jax.ShapeDtypeStruct((_NC, n, 16), jnp.float32),
        mesh=mesh,
        compiler_params=pltpu.CompilerParams(use_tc_tiling_on_sc=False),
        scratch_types=[
            pltpu.VMEM((_K,), jnp.int32),
            pltpu.VMEM((_K,), jnp.float32),
            pltpu.VMEM((_K, 16), jnp.float32),
            pltpu.VMEM_SHARED((n, 16), jnp.float32),
        ],
    )
    def kern(dst_hbm, w_hbm, out_hbm, dstv, wv, rows, deg_sh):
        c = lax.axis_index("c")
        s = lax.axis_index("s")
        wid = c * _NS + s
        zvec = jnp.zeros((16,), jnp.float32)

        @pl.loop(0, _K)
        def _zero_rows(i):
            rows[i, :] = zvec

        row0 = s * rpt
        for k in range(nz):
            pltpu.sync_copy(rows.at[pl.ds(0, _K)], deg_sh.at[pl.ds(row0 + k * _K, _K)])
        if rem:
            pltpu.sync_copy(rows.at[pl.ds(0, rem)], deg_sh.at[pl.ds(row0 + nz * _K, rem)])
        if tail:
            @pl.when(s == _NS - 1)
            def _zero_tail():
                pltpu.sync_copy(rows.at[pl.ds(0, tail)], deg_sh.at[pl.ds(n - tail, tail)])
        plsc.subcore_barrier()

        @pl.loop(0, nchunks)
        def _chunk(j):
            base = wid * ew + j * _K
            pltpu.sync_copy(dst_hbm.at[pl.ds(base, _K)], dstv)
            pltpu.sync_copy(w_hbm.at[pl.ds(base, _K)], wv)
            @pl.loop(0, _K // 16)
            def _fill(g):
                wvec = wv[pl.ds(g * 16, 16)]
                for l in range(16):
                    rows[g * 16 + l, :] = jnp.full((16,), wvec[l], jnp.float32)
            pltpu.sync_copy(rows, deg_sh.at[dstv], add=True)

        plsc.subcore_barrier()
        pltpu.sync_copy(deg_sh.at[pl.ds(row0, rpt)], out_hbm.at[c].at[pl.ds(row0, rpt)])
        if tail:
            @pl.when(s == _NS - 1)
            def _wb_tail():
                pltpu.sync_copy(deg_sh.at[pl.ds(n - tail, tail)],
                                out_hbm.at[c].at[pl.ds(n - tail, tail)])

    return kern


def _sc_scatter(n, e, f):
    nw = _NC * _NS
    ew = e // nw
    assert e % nw == 0 and ew % _K == 0 and n % _NS == 0 and f % 16 == 0
    nchunks = ew // _K
    rpt = (n // 8 // _NS) * 8
    tail = n - rpt * _NS
    nz, rem = rpt // _K, rpt % _K
    mesh = plsc.VectorSubcoreMesh(core_axis_name="c", subcore_axis_name="s")

    @functools.partial(
        pl.kernel,
        out_type=jax.ShapeDtypeStruct((_NC, n, f), jnp.float32),
        mesh=mesh,
        compiler_params=pltpu.CompilerParams(use_tc_tiling_on_sc=False),
        scratch_types=[
            pltpu.VMEM((_K,), jnp.int32),
            pltpu.VMEM((_K,), jnp.int32),
            pltpu.VMEM((_K,), jnp.float32),
            pltpu.VMEM((_K, f), jnp.float32),
            pltpu.VMEM_SHARED((n, f), jnp.float32),
            pltpu.SemaphoreType.DMA,
        ],
    )
    def kern(y_hbm, src_hbm, dst_hbm, w_hbm, out_hbm, srcv, dstv, wv, rows, acc, sem):
        c = lax.axis_index("c")
        s = lax.axis_index("s")
        wid = c * _NS + s
        zvec = jnp.zeros((16,), jnp.float32)

        @pl.loop(0, _K)
        def _zero_rows(i):
            for g in range(f // 16):
                rows[i, pl.ds(g * 16, 16)] = zvec

        row0 = s * rpt
        for k in range(nz):
            pltpu.sync_copy(rows.at[pl.ds(0, _K)], acc.at[pl.ds(row0 + k * _K, _K)])
        if rem:
            pltpu.sync_copy(rows.at[pl.ds(0, rem)], acc.at[pl.ds(row0 + nz * _K, rem)])
        if tail:
            @pl.when(s == _NS - 1)
            def _zero_tail():
                pltpu.sync_copy(rows.at[pl.ds(0, tail)], acc.at[pl.ds(n - tail, tail)])
        plsc.subcore_barrier()

        @pl.loop(0, nchunks)
        def _chunk(j):
            base = wid * ew + j * _K
            pltpu.sync_copy(src_hbm.at[pl.ds(base, _K)], srcv)
            pltpu.sync_copy(dst_hbm.at[pl.ds(base, _K)], dstv)
            pltpu.sync_copy(w_hbm.at[pl.ds(base, _K)], wv)
            pltpu.async_copy(y_hbm.at[srcv], rows, sem).wait()

            @pl.loop(0, _K // 16)
            def _scale(g):
                wvec = wv[pl.ds(g * 16, 16)]
                for l in range(16):
                    wi = wvec[l]
                    i = g * 16 + l
                    for fb in range(f // 16):
                        rows[i, pl.ds(fb * 16, 16)] = rows[i, pl.ds(fb * 16, 16)] * wi

            pltpu.sync_copy(rows, acc.at[dstv], add=True)

        plsc.subcore_barrier()
        pltpu.sync_copy(acc.at[pl.ds(row0, rpt)], out_hbm.at[c].at[pl.ds(row0, rpt)])
        if tail:
            @pl.when(s == _NS - 1)
            def _wb_tail():
                pltpu.sync_copy(acc.at[pl.ds(n - tail, tail)],
                                out_hbm.at[c].at[pl.ds(n - tail, tail)])

    return kern


def _tc_mm1(n, fin, fh, blk):
    def body(deg_ref, x_ref, w_ref, y_ref, dinv_ref):
        d = deg_ref[0, :, 0:1] + deg_ref[1, :, 0:1] + 1.0
        dinv = jnp.where(d > 0, lax.rsqrt(d), 0.0)
        xw = jnp.dot(x_ref[...], w_ref[...], preferred_element_type=jnp.float32,
                     precision=lax.Precision.HIGHEST)
        y_ref[...] = dinv * xw
        dinv_ref[...] = dinv

    return pl.pallas_call(
        body,
        grid=(n // blk,),
        in_specs=[
            pl.BlockSpec((_NC, blk, 16), lambda i: (0, i, 0)),
            pl.BlockSpec((blk, fin), lambda i: (i, 0)),
            pl.BlockSpec((fin, fh), lambda i: (0, 0)),
        ],
        out_specs=[
            pl.BlockSpec((blk, fh), lambda i: (i, 0)),
            pl.BlockSpec((blk, 1), lambda i: (i, 0)),
        ],
        out_shape=[
            jax.ShapeDtypeStruct((n, fh), jnp.float32),
            jax.ShapeDtypeStruct((n, 1), jnp.float32),
        ],
    )


def _tc_mm2(n, fh, fo, blk):
    def body(acc_ref, y1_ref, dinv_ref, b1_ref, w2_ref, y2_ref):
        dinv = dinv_ref[...]
        a = acc_ref[0] + acc_ref[1] + y1_ref[...]
        h = jnp.maximum(dinv * a + b1_ref[...], 0.0)
        y2_ref[...] = dinv * jnp.dot(h, w2_ref[...], preferred_element_type=jnp.float32,
                                     precision=lax.Precision.HIGHEST)

    return pl.pallas_call(
        body,
        grid=(n // blk,),
        in_specs=[
            pl.BlockSpec((_NC, blk, fh), lambda i: (0, i, 0)),
            pl.BlockSpec((blk, fh), lambda i: (i, 0)),
            pl.BlockSpec((blk, 1), lambda i: (i, 0)),
            pl.BlockSpec((1, fh), lambda i: (0, 0)),
            pl.BlockSpec((fh, fo), lambda i: (0, 0)),
        ],
        out_specs=pl.BlockSpec((blk, fo), lambda i: (i, 0)),
        out_shape=jax.ShapeDtypeStruct((n, fo), jnp.float32),
    )


def _tc_fin(n, fo, blk):
    def body(acc_ref, y2_ref, dinv_ref, b2_ref, out_ref):
        out_ref[...] = dinv_ref[...] * (acc_ref[0] + acc_ref[1] + y2_ref[...]) + b2_ref[...]

    return pl.pallas_call(
        body,
        grid=(n // blk,),
        in_specs=[
            pl.BlockSpec((_NC, blk, fo), lambda i: (0, i, 0)),
            pl.BlockSpec((blk, fo), lambda i: (i, 0)),
            pl.BlockSpec((blk, 1), lambda i: (i, 0)),
            pl.BlockSpec((1, fo), lambda i: (0, 0)),
        ],
        out_specs=pl.BlockSpec((blk, fo), lambda i: (i, 0)),
        out_shape=jax.ShapeDtypeStruct((n, fo), jnp.float32),
    )


def kernel(in_feat, edge_index, edge_weight, W1, b1, W2, b2):
    n, fin = in_feat.shape
    e = edge_index.shape[1]
    fh = W1.shape[1]
    fo = W2.shape[1]
    blk = 1000
    assert n % blk == 0

    src = edge_index[0].astype(jnp.int32)
    dst = edge_index[1].astype(jnp.int32)
    w = edge_weight.astype(jnp.float32)

    deg2 = _sc_degree(n, e)(dst, w)
    y1, dinv = _tc_mm1(n, fin, fh, blk)(deg2, in_feat.astype(jnp.float32), W1)
    acc1 = _sc_scatter(n, e, fh)(y1, src, dst, w)
    y2 = _tc_mm2(n, fh, fo, blk)(acc1, y1, dinv, b1.reshape(1, fh), W2)
    acc2 = _sc_scatter(n, e, fo)(y2, src, dst, w)
    out = _tc_fin(n, fo, blk)(acc2, y2, dinv, b2.reshape(1, fo))
    return out


# speedup vs baseline: 13.9479x; 1.5469x over previous
import functools

import jax
import jax.numpy as jnp
from jax import lax
from jax.experimental import pallas as pl
from jax.experimental.pallas import tpu as pltpu
from jax.experimental.pallas import tpu_sc as plsc

_NC = 2
_NS = 16
_K = 80


def _sc_degree(n, e):
    nw = _NC * _NS
    ew = e // nw
    assert e % nw == 0 and ew % _K == 0 and n % _NS == 0
    nchunks = ew // _K
    rpt = (n // 8 // _NS) * 8
    tail = n - rpt * _NS
    nz, rem = rpt // _K, rpt % _K
    mesh = plsc.VectorSubcoreMesh(core_axis_name="c", subcore_axis_name="s")

    @functools.partial(
        pl.kernel,
        out_type=jax.ShapeDtypeStruct((_NC, n, 16), jnp.float32),
        mesh=mesh,
        compiler_params=pltpu.CompilerParams(use_tc_tiling_on_sc=False),
        scratch_types=[
            pltpu.VMEM((_K,), jnp.int32),
            pltpu.VMEM((_K,), jnp.float32),
            pltpu.VMEM((_K, 16), jnp.float32),
            pltpu.VMEM_SHARED((n, 16), jnp.float32),
        ],
    )
    def kern(dst_hbm, w_hbm, out_hbm, dstv, wv, rows, deg_sh):
        c = lax.axis_index("c")
        s = lax.axis_index("s")
        wid = c * _NS + s
        zvec = jnp.zeros((16,), jnp.float32)

        @pl.loop(0, _K)
        def _zero_rows(i):
            rows[i, :] = zvec

        row0 = s * rpt
        for k in range(nz):
            pltpu.sync_copy(rows.at[pl.ds(0, _K)], deg_sh.at[pl.ds(row0 + k * _K, _K)])
        if rem:
            pltpu.sync_copy(rows.at[pl.ds(0, rem)], deg_sh.at[pl.ds(row0 + nz * _K, rem)])
        if tail:
            @pl.when(s == _NS - 1)
            def _zero_tail():
                pltpu.sync_copy(rows.at[pl.ds(0, tail)], deg_sh.at[pl.ds(n - tail, tail)])
        plsc.subcore_barrier()

        @pl.loop(0, nchunks)
        def _chunk(j):
            base = wid * ew + j * _K
            pltpu.sync_copy(dst_hbm.at[pl.ds(base, _K)], dstv)
            pltpu.sync_copy(w_hbm.at[pl.ds(base, _K)], wv)
            @pl.loop(0, _K // 16)
            def _fill(g):
                wvec = wv[pl.ds(g * 16, 16)]
                for l in range(16):
                    rows[g * 16 + l, :] = jnp.full((16,), wvec[l], jnp.float32)
            pltpu.sync_copy(rows, deg_sh.at[dstv], add=True)

        plsc.subcore_barrier()
        pltpu.sync_copy(deg_sh.at[pl.ds(row0, rpt)], out_hbm.at[c].at[pl.ds(row0, rpt)])
        if tail:
            @pl.when(s == _NS - 1)
            def _wb_tail():
                pltpu.sync_copy(deg_sh.at[pl.ds(n - tail, tail)],
                                out_hbm.at[c].at[pl.ds(n - tail, tail)])

    return kern


def _sc_scatter(n, e, f):
    nw = _NC * _NS
    ew = e // nw
    assert e % nw == 0 and ew % _K == 0 and n % _NS == 0 and f % 16 == 0
    nchunks = ew // _K
    assert nchunks >= 3 and nchunks % 2 == 1
    rpt = (n // 8 // _NS) * 8
    tail = n - rpt * _NS
    nz, rem = rpt // _K, rpt % _K
    mesh = plsc.VectorSubcoreMesh(core_axis_name="c", subcore_axis_name="s")

    @functools.partial(
        pl.kernel,
        out_type=jax.ShapeDtypeStruct((_NC, n, f), jnp.float32),
        mesh=mesh,
        compiler_params=pltpu.CompilerParams(use_tc_tiling_on_sc=False),
        scratch_types=[
            pltpu.VMEM((_K,), jnp.int32),
            pltpu.VMEM((_K,), jnp.int32),
            pltpu.VMEM((_K,), jnp.int32),
            pltpu.VMEM((_K,), jnp.int32),
            pltpu.VMEM((_K,), jnp.float32),
            pltpu.VMEM((_K,), jnp.float32),
            pltpu.VMEM((_K,), jnp.int32),
            pltpu.VMEM((_K,), jnp.int32),
            pltpu.VMEM((_K, f), jnp.float32),
            pltpu.VMEM((_K, f), jnp.float32),
            pltpu.VMEM_SHARED((n, f), jnp.float32),
            pltpu.SemaphoreType.DMA,
            pltpu.SemaphoreType.DMA,
            pltpu.SemaphoreType.DMA,
            pltpu.SemaphoreType.DMA,
            pltpu.SemaphoreType.DMA,
            pltpu.SemaphoreType.DMA,
        ],
    )
    def kern(y_hbm, src_hbm, dst_hbm, w_hbm, out_hbm,
             srcv0, srcv1, dstv0, dstv1, wv0, wv1, dsc0, dsc1, rows0, rows1,
             acc, isem0, isem1, gsem0, gsem1, ssem0, ssem1):
        c = lax.axis_index("c")
        s = lax.axis_index("s")
        wid = c * _NS + s
        zvec = jnp.zeros((16,), jnp.float32)
        bufs = [
            (srcv0, dstv0, wv0, dsc0, rows0, isem0, gsem0, ssem0),
            (srcv1, dstv1, wv1, dsc1, rows1, isem1, gsem1, ssem1),
        ]

        @pl.loop(0, _K)
        def _zero_rows(i):
            for g in range(f // 16):
                rows0[i, pl.ds(g * 16, 16)] = zvec

        row0 = s * rpt
        for k in range(nz):
            pltpu.sync_copy(rows0.at[pl.ds(0, _K)], acc.at[pl.ds(row0 + k * _K, _K)])
        if rem:
            pltpu.sync_copy(rows0.at[pl.ds(0, rem)], acc.at[pl.ds(row0 + nz * _K, rem)])
        if tail:
            @pl.when(s == _NS - 1)
            def _zero_tail():
                pltpu.sync_copy(rows0.at[pl.ds(0, tail)], acc.at[pl.ds(n - tail, tail)])
        plsc.subcore_barrier()

        def ibase(j):
            return wid * ew + j * _K

        def issue_idx(j, p):
            srcv, dstv, wv, _, _, isem, _, _ = bufs[p]
            b = ibase(j)
            pltpu.async_copy(src_hbm.at[pl.ds(b, _K)], srcv, isem)
            pltpu.async_copy(dst_hbm.at[pl.ds(b, _K)], dstv, isem)
            pltpu.async_copy(w_hbm.at[pl.ds(b, _K)], wv, isem)

        def wait_idx(j, p):
            srcv, dstv, wv, _, _, isem, _, _ = bufs[p]
            b = ibase(j)
            pltpu.make_async_copy(src_hbm.at[pl.ds(b, _K)], srcv, isem).wait()
            pltpu.make_async_copy(dst_hbm.at[pl.ds(b, _K)], dstv, isem).wait()
            pltpu.make_async_copy(w_hbm.at[pl.ds(b, _K)], wv, isem).wait()

        def issue_gather(p):
            srcv, _, _, _, rows, _, gsem, _ = bufs[p]
            pltpu.async_copy(y_hbm.at[srcv], rows, gsem)

        def wait_gather(p):
            srcv, _, _, _, rows, _, gsem, _ = bufs[p]
            pltpu.make_async_copy(y_hbm.at[srcv], rows, gsem).wait()

        def issue_scat(p):
            _, _, _, dsc, rows, _, _, ssem = bufs[p]
            pltpu.async_copy(rows, acc.at[dsc], ssem, add=True)

        def drain_scat(p):
            _, _, _, dsc, rows, _, _, ssem = bufs[p]
            pltpu.make_async_copy(rows, acc.at[dsc], ssem).wait()

        def half(j, p, last):
            srcv, dstv, wv, dsc, rows, isem, gsem, ssem = bufs[p]
            pn = 1 - p
            wait_gather(p)
            for g in range(_K // 16):
                dsc[pl.ds(g * 16, 16)] = dstv[pl.ds(g * 16, 16)]

            @pl.loop(0, _K // 16)
            def _scale(g):
                wvec = wv[pl.ds(g * 16, 16)]
                for l in range(16):
                    wi = wvec[l]
                    i = g * 16 + l
                    for fb in range(f // 16):
                        rows[i, pl.ds(fb * 16, 16)] = rows[i, pl.ds(fb * 16, 16)] * wi

            if not last:
                @pl.when(j + 2 < nchunks)
                def _pf2():
                    issue_idx(j + 2, p)

                @pl.when(j >= 1)
                def _drain_prev():
                    drain_scat(pn)
            else:
                drain_scat(pn)
            issue_scat(p)
            if not last:
                wait_idx(j + 1, pn)
                issue_gather(pn)

        issue_idx(0, 0)
        issue_idx(1, 1)
        wait_idx(0, 0)
        issue_gather(0)

        @pl.loop(0, nchunks // 2)
        def _pair(t):
            half(2 * t, 0, False)
            half(2 * t + 1, 1, False)

        half(nchunks - 1, 0, True)
        drain_scat(0)

        plsc.subcore_barrier()
        pltpu.sync_copy(acc.at[pl.ds(row0, rpt)], out_hbm.at[c].at[pl.ds(row0, rpt)])
        if tail:
            @pl.when(s == _NS - 1)
            def _wb_tail():
                pltpu.sync_copy(acc.at[pl.ds(n - tail, tail)],
                                out_hbm.at[c].at[pl.ds(n - tail, tail)])

    return kern


def _tc_mm1(n, fin, fh, blk):
    def body(deg_ref, x_ref, w_ref, y_ref, dinv_ref):
        d = deg_ref[0, :, 0:1] + deg_ref[1, :, 0:1] + 1.0
        dinv = jnp.where(d > 0, lax.rsqrt(d), 0.0)
        xw = jnp.dot(x_ref[...], w_ref[...], preferred_element_type=jnp.float32,
                     precision=lax.Precision.HIGHEST)
        y_ref[...] = dinv * xw
        dinv_ref[...] = dinv

    return pl.pallas_call(
        body,
        grid=(n // blk,),
        in_specs=[
            pl.BlockSpec((_NC, blk, 16), lambda i: (0, i, 0)),
            pl.BlockSpec((blk, fin), lambda i: (i, 0)),
            pl.BlockSpec((fin, fh), lambda i: (0, 0)),
        ],
        out_specs=[
            pl.BlockSpec((blk, fh), lambda i: (i, 0)),
            pl.BlockSpec((blk, 1), lambda i: (i, 0)),
        ],
        out_shape=[
            jax.ShapeDtypeStruct((n, fh), jnp.float32),
            jax.ShapeDtypeStruct((n, 1), jnp.float32),
        ],
    )


def _tc_mm2(n, fh, fo, blk):
    def body(acc_ref, y1_ref, dinv_ref, b1_ref, w2_ref, y2_ref):
        dinv = dinv_ref[...]
        a = acc_ref[0] + acc_ref[1] + y1_ref[...]
        h = jnp.maximum(dinv * a + b1_ref[...], 0.0)
        y2_ref[...] = dinv * jnp.dot(h, w2_ref[...], preferred_element_type=jnp.float32,
                                     precision=lax.Precision.HIGHEST)

    return pl.pallas_call(
        body,
        grid=(n // blk,),
        in_specs=[
            pl.BlockSpec((_NC, blk, fh), lambda i: (0, i, 0)),
            pl.BlockSpec((blk, fh), lambda i: (i, 0)),
            pl.BlockSpec((blk, 1), lambda i: (i, 0)),
            pl.BlockSpec((1, fh), lambda i: (0, 0)),
            pl.BlockSpec((fh, fo), lambda i: (0, 0)),
        ],
        out_specs=pl.BlockSpec((blk, fo), lambda i: (i, 0)),
        out_shape=jax.ShapeDtypeStruct((n, fo), jnp.float32),
    )


def _tc_fin(n, fo, blk):
    def body(acc_ref, y2_ref, dinv_ref, b2_ref, out_ref):
        out_ref[...] = dinv_ref[...] * (acc_ref[0] + acc_ref[1] + y2_ref[...]) + b2_ref[...]

    return pl.pallas_call(
        body,
        grid=(n // blk,),
        in_specs=[
            pl.BlockSpec((_NC, blk, fo), lambda i: (0, i, 0)),
            pl.BlockSpec((blk, fo), lambda i: (i, 0)),
            pl.BlockSpec((blk, 1), lambda i: (i, 0)),
            pl.BlockSpec((1, fo), lambda i: (0, 0)),
        ],
        out_specs=pl.BlockSpec((blk, fo), lambda i: (i, 0)),
        out_shape=jax.ShapeDtypeStruct((n, fo), jnp.float32),
    )


def kernel(in_feat, edge_index, edge_weight, W1, b1, W2, b2):
    n, fin = in_feat.shape
    e = edge_index.shape[1]
    fh = W1.shape[1]
    fo = W2.shape[1]
    blk = 1000
    assert n % blk == 0

    src = edge_index[0].astype(jnp.int32)
    dst = edge_index[1].astype(jnp.int32)
    w = edge_weight.astype(jnp.float32)

    deg2 = _sc_degree(n, e)(dst, w)
    y1, dinv = _tc_mm1(n, fin, fh, blk)(deg2, in_feat.astype(jnp.float32), W1)
    acc1 = _sc_scatter(n, e, fh)(y1, src, dst, w)
    y2 = _tc_mm2(n, fh, fo, blk)(acc1, y1, dinv, b1.reshape(1, fh), W2)
    acc2 = _sc_scatter(n, e, fo)(y2, src, dst, w)
    out = _tc_fin(n, fo, blk)(acc2, y2, dinv, b2.reshape(1, fo))
    return out


# speedup vs baseline: 16.0874x; 1.1534x over previous
import functools

import jax
import jax.numpy as jnp
from jax import lax
from jax.experimental import pallas as pl
from jax.experimental.pallas import tpu as pltpu
from jax.experimental.pallas import tpu_sc as plsc

_NC = 2
_NS = 16
_K = 80


def _sc_degree(n, e):
    nw = _NC * _NS
    ew = e // nw
    assert e % nw == 0 and ew % _K == 0 and n % _NS == 0
    nchunks = ew // _K
    rpt = (n // 8 // _NS) * 8
    tail = n - rpt * _NS
    nz, rem = rpt // _K, rpt % _K
    mesh = plsc.VectorSubcoreMesh(core_axis_name="c", subcore_axis_name="s")

    assert nchunks >= 3 and nchunks % 2 == 1

    @functools.partial(
        pl.kernel,
        out_type=jax.ShapeDtypeStruct((_NC, n, 16), jnp.float32),
        mesh=mesh,
        compiler_params=pltpu.CompilerParams(use_tc_tiling_on_sc=False),
        scratch_types=[
            pltpu.VMEM((_K,), jnp.int32),
            pltpu.VMEM((_K,), jnp.int32),
            pltpu.VMEM((_K,), jnp.float32),
            pltpu.VMEM((_K,), jnp.float32),
            pltpu.VMEM((_K,), jnp.int32),
            pltpu.VMEM((_K,), jnp.int32),
            pltpu.VMEM((_K, 16), jnp.float32),
            pltpu.VMEM((_K, 16), jnp.float32),
            pltpu.VMEM_SHARED((n, 16), jnp.float32),
            pltpu.SemaphoreType.DMA,
            pltpu.SemaphoreType.DMA,
            pltpu.SemaphoreType.DMA,
            pltpu.SemaphoreType.DMA,
        ],
    )
    def kern(dst_hbm, w_hbm, out_hbm, dstv0, dstv1, wv0, wv1, dsc0, dsc1,
             rows0, rows1, deg_sh, isem0, isem1, ssem0, ssem1):
        c = lax.axis_index("c")
        s = lax.axis_index("s")
        wid = c * _NS + s
        zvec = jnp.zeros((16,), jnp.float32)
        bufs = [
            (dstv0, wv0, dsc0, rows0, isem0, ssem0),
            (dstv1, wv1, dsc1, rows1, isem1, ssem1),
        ]

        @pl.loop(0, _K)
        def _zero_rows(i):
            rows0[i, :] = zvec

        row0 = s * rpt
        for k in range(nz):
            pltpu.sync_copy(rows0.at[pl.ds(0, _K)], deg_sh.at[pl.ds(row0 + k * _K, _K)])
        if rem:
            pltpu.sync_copy(rows0.at[pl.ds(0, rem)], deg_sh.at[pl.ds(row0 + nz * _K, rem)])
        if tail:
            @pl.when(s == _NS - 1)
            def _zero_tail():
                pltpu.sync_copy(rows0.at[pl.ds(0, tail)], deg_sh.at[pl.ds(n - tail, tail)])
        plsc.subcore_barrier()

        def ibase(j):
            return wid * ew + j * _K

        def issue_idx(j, p):
            dstv, wv, _, _, isem, _ = bufs[p]
            b = ibase(j)
            pltpu.async_copy(dst_hbm.at[pl.ds(b, _K)], dstv, isem)
            pltpu.async_copy(w_hbm.at[pl.ds(b, _K)], wv, isem)

        def wait_idx(j, p):
            dstv, wv, _, _, isem, _ = bufs[p]
            b = ibase(j)
            pltpu.make_async_copy(dst_hbm.at[pl.ds(b, _K)], dstv, isem).wait()
            pltpu.make_async_copy(w_hbm.at[pl.ds(b, _K)], wv, isem).wait()

        def issue_scat(p):
            _, _, dsc, rows, _, ssem = bufs[p]
            pltpu.async_copy(rows, deg_sh.at[dsc], ssem, add=True)

        def drain_scat(p):
            _, _, dsc, rows, _, ssem = bufs[p]
            pltpu.make_async_copy(rows, deg_sh.at[dsc], ssem).wait()

        def half(j, p, last):
            dstv, wv, dsc, rows, isem, ssem = bufs[p]
            wait_idx(j, p)
            if last:
                drain_scat(p)
            else:
                @pl.when(j >= 2)
                def _drain_prev2():
                    drain_scat(p)
            for g in range(_K // 16):
                dsc[pl.ds(g * 16, 16)] = dstv[pl.ds(g * 16, 16)]

            @pl.loop(0, _K // 16)
            def _fill(g):
                wvec = wv[pl.ds(g * 16, 16)]
                for l in range(16):
                    rows[g * 16 + l, :] = jnp.full((16,), wvec[l], jnp.float32)

            if not last:
                @pl.when(j + 2 < nchunks)
                def _pf2():
                    issue_idx(j + 2, p)
            issue_scat(p)

        issue_idx(0, 0)
        issue_idx(1, 1)

        @pl.loop(0, nchunks // 2)
        def _pair(t):
            half(2 * t, 0, False)
            half(2 * t + 1, 1, False)

        half(nchunks - 1, 0, True)
        drain_scat(1)
        drain_scat(0)

        plsc.subcore_barrier()
        pltpu.sync_copy(deg_sh.at[pl.ds(row0, rpt)], out_hbm.at[c].at[pl.ds(row0, rpt)])
        if tail:
            @pl.when(s == _NS - 1)
            def _wb_tail():
                pltpu.sync_copy(deg_sh.at[pl.ds(n - tail, tail)],
                                out_hbm.at[c].at[pl.ds(n - tail, tail)])

    return kern


def _sc_scatter(n, e, f):
    nw = _NC * _NS
    ew = e // nw
    assert e % nw == 0 and ew % _K == 0 and n % _NS == 0 and f % 16 == 0
    nchunks = ew // _K
    assert nchunks >= 3 and nchunks % 2 == 1
    rpt = (n // 8 // _NS) * 8
    tail = n - rpt * _NS
    nz, rem = rpt // _K, rpt % _K
    mesh = plsc.VectorSubcoreMesh(core_axis_name="c", subcore_axis_name="s")

    @functools.partial(
        pl.kernel,
        out_type=jax.ShapeDtypeStruct((_NC, n, f), jnp.float32),
        mesh=mesh,
        compiler_params=pltpu.CompilerParams(use_tc_tiling_on_sc=False),
        scratch_types=[
            pltpu.VMEM((_K,), jnp.int32),
            pltpu.VMEM((_K,), jnp.int32),
            pltpu.VMEM((_K,), jnp.int32),
            pltpu.VMEM((_K,), jnp.int32),
            pltpu.VMEM((_K,), jnp.float32),
            pltpu.VMEM((_K,), jnp.float32),
            pltpu.VMEM((_K,), jnp.int32),
            pltpu.VMEM((_K,), jnp.int32),
            pltpu.VMEM((_K, f), jnp.float32),
            pltpu.VMEM((_K, f), jnp.float32),
            pltpu.VMEM_SHARED((n, f), jnp.float32),
            pltpu.SemaphoreType.DMA,
            pltpu.SemaphoreType.DMA,
            pltpu.SemaphoreType.DMA,
            pltpu.SemaphoreType.DMA,
            pltpu.SemaphoreType.DMA,
            pltpu.SemaphoreType.DMA,
        ],
    )
    def kern(y_hbm, src_hbm, dst_hbm, w_hbm, out_hbm,
             srcv0, srcv1, dstv0, dstv1, wv0, wv1, dsc0, dsc1, rows0, rows1,
             acc, isem0, isem1, gsem0, gsem1, ssem0, ssem1):
        c = lax.axis_index("c")
        s = lax.axis_index("s")
        wid = c * _NS + s
        zvec = jnp.zeros((16,), jnp.float32)
        bufs = [
            (srcv0, dstv0, wv0, dsc0, rows0, isem0, gsem0, ssem0),
            (srcv1, dstv1, wv1, dsc1, rows1, isem1, gsem1, ssem1),
        ]

        @pl.loop(0, _K)
        def _zero_rows(i):
            for g in range(f // 16):
                rows0[i, pl.ds(g * 16, 16)] = zvec

        row0 = s * rpt
        for k in range(nz):
            pltpu.sync_copy(rows0.at[pl.ds(0, _K)], acc.at[pl.ds(row0 + k * _K, _K)])
        if rem:
            pltpu.sync_copy(rows0.at[pl.ds(0, rem)], acc.at[pl.ds(row0 + nz * _K, rem)])
        if tail:
            @pl.when(s == _NS - 1)
            def _zero_tail():
                pltpu.sync_copy(rows0.at[pl.ds(0, tail)], acc.at[pl.ds(n - tail, tail)])
        plsc.subcore_barrier()

        def ibase(j):
            return wid * ew + j * _K

        def issue_idx(j, p):
            srcv, dstv, wv, _, _, isem, _, _ = bufs[p]
            b = ibase(j)
            pltpu.async_copy(src_hbm.at[pl.ds(b, _K)], srcv, isem)
            pltpu.async_copy(dst_hbm.at[pl.ds(b, _K)], dstv, isem)
            pltpu.async_copy(w_hbm.at[pl.ds(b, _K)], wv, isem)

        def wait_idx(j, p):
            srcv, dstv, wv, _, _, isem, _, _ = bufs[p]
            b = ibase(j)
            pltpu.make_async_copy(src_hbm.at[pl.ds(b, _K)], srcv, isem).wait()
            pltpu.make_async_copy(dst_hbm.at[pl.ds(b, _K)], dstv, isem).wait()
            pltpu.make_async_copy(w_hbm.at[pl.ds(b, _K)], wv, isem).wait()

        def issue_gather(p):
            srcv, _, _, _, rows, _, gsem, _ = bufs[p]
            pltpu.async_copy(y_hbm.at[srcv], rows, gsem)

        def wait_gather(p):
            srcv, _, _, _, rows, _, gsem, _ = bufs[p]
            pltpu.make_async_copy(y_hbm.at[srcv], rows, gsem).wait()

        def issue_scat(p):
            _, _, _, dsc, rows, _, _, ssem = bufs[p]
            pltpu.async_copy(rows, acc.at[dsc], ssem, add=True)

        def drain_scat(p):
            _, _, _, dsc, rows, _, _, ssem = bufs[p]
            pltpu.make_async_copy(rows, acc.at[dsc], ssem).wait()

        def half(j, p, last):
            srcv, dstv, wv, dsc, rows, isem, gsem, ssem = bufs[p]
            pn = 1 - p
            wait_gather(p)
            for g in range(_K // 16):
                dsc[pl.ds(g * 16, 16)] = dstv[pl.ds(g * 16, 16)]

            @pl.loop(0, _K // 16)
            def _scale(g):
                wvec = wv[pl.ds(g * 16, 16)]
                for l in range(16):
                    wi = wvec[l]
                    i = g * 16 + l
                    for fb in range(f // 16):
                        rows[i, pl.ds(fb * 16, 16)] = rows[i, pl.ds(fb * 16, 16)] * wi

            if not last:
                @pl.when(j + 2 < nchunks)
                def _pf2():
                    issue_idx(j + 2, p)

                @pl.when(j >= 1)
                def _drain_prev():
                    drain_scat(pn)
            else:
                drain_scat(pn)
            issue_scat(p)
            if not last:
                wait_idx(j + 1, pn)
                issue_gather(pn)

        issue_idx(0, 0)
        issue_idx(1, 1)
        wait_idx(0, 0)
        issue_gather(0)

        @pl.loop(0, nchunks // 2)
        def _pair(t):
            half(2 * t, 0, False)
            half(2 * t + 1, 1, False)

        half(nchunks - 1, 0, True)
        drain_scat(0)

        plsc.subcore_barrier()
        pltpu.sync_copy(acc.at[pl.ds(row0, rpt)], out_hbm.at[c].at[pl.ds(row0, rpt)])
        if tail:
            @pl.when(s == _NS - 1)
            def _wb_tail():
                pltpu.sync_copy(acc.at[pl.ds(n - tail, tail)],
                                out_hbm.at[c].at[pl.ds(n - tail, tail)])

    return kern


def _tc_mm1(n, fin, fh, blk):
    def body(deg_ref, x_ref, w_ref, y_ref, dinv_ref):
        d = deg_ref[0, :, 0:1] + deg_ref[1, :, 0:1] + 1.0
        dinv = jnp.where(d > 0, lax.rsqrt(d), 0.0)
        xw = jnp.dot(x_ref[...], w_ref[...], preferred_element_type=jnp.float32,
                     precision=lax.Precision.HIGHEST)
        y_ref[...] = dinv * xw
        dinv_ref[...] = dinv

    return pl.pallas_call(
        body,
        grid=(n // blk,),
        in_specs=[
            pl.BlockSpec((_NC, blk, 16), lambda i: (0, i, 0)),
            pl.BlockSpec((blk, fin), lambda i: (i, 0)),
            pl.BlockSpec((fin, fh), lambda i: (0, 0)),
        ],
        out_specs=[
            pl.BlockSpec((blk, fh), lambda i: (i, 0)),
            pl.BlockSpec((blk, 1), lambda i: (i, 0)),
        ],
        out_shape=[
            jax.ShapeDtypeStruct((n, fh), jnp.float32),
            jax.ShapeDtypeStruct((n, 1), jnp.float32),
        ],
    )


def _tc_mm2(n, fh, fo, blk):
    def body(acc_ref, y1_ref, dinv_ref, b1_ref, w2_ref, y2_ref):
        dinv = dinv_ref[...]
        a = acc_ref[0] + acc_ref[1] + y1_ref[...]
        h = jnp.maximum(dinv * a + b1_ref[...], 0.0)
        y2_ref[...] = dinv * jnp.dot(h, w2_ref[...], preferred_element_type=jnp.float32,
                                     precision=lax.Precision.HIGHEST)

    return pl.pallas_call(
        body,
        grid=(n // blk,),
        in_specs=[
            pl.BlockSpec((_NC, blk, fh), lambda i: (0, i, 0)),
            pl.BlockSpec((blk, fh), lambda i: (i, 0)),
            pl.BlockSpec((blk, 1), lambda i: (i, 0)),
            pl.BlockSpec((1, fh), lambda i: (0, 0)),
            pl.BlockSpec((fh, fo), lambda i: (0, 0)),
        ],
        out_specs=pl.BlockSpec((blk, fo), lambda i: (i, 0)),
        out_shape=jax.ShapeDtypeStruct((n, fo), jnp.float32),
    )


def _tc_fin(n, fo, blk):
    def body(acc_ref, y2_ref, dinv_ref, b2_ref, out_ref):
        out_ref[...] = dinv_ref[...] * (acc_ref[0] + acc_ref[1] + y2_ref[...]) + b2_ref[...]

    return pl.pallas_call(
        body,
        grid=(n // blk,),
        in_specs=[
            pl.BlockSpec((_NC, blk, fo), lambda i: (0, i, 0)),
            pl.BlockSpec((blk, fo), lambda i: (i, 0)),
            pl.BlockSpec((blk, 1), lambda i: (i, 0)),
            pl.BlockSpec((1, fo), lambda i: (0, 0)),
        ],
        out_specs=pl.BlockSpec((blk, fo), lambda i: (i, 0)),
        out_shape=jax.ShapeDtypeStruct((n, fo), jnp.float32),
    )


def kernel(in_feat, edge_index, edge_weight, W1, b1, W2, b2):
    n, fin = in_feat.shape
    e = edge_index.shape[1]
    fh = W1.shape[1]
    fo = W2.shape[1]
    blk = 1000
    assert n % blk == 0

    src = edge_index[0].astype(jnp.int32)
    dst = edge_index[1].astype(jnp.int32)
    w = edge_weight.astype(jnp.float32)

    deg2 = _sc_degree(n, e)(dst, w)
    y1, dinv = _tc_mm1(n, fin, fh, blk)(deg2, in_feat.astype(jnp.float32), W1)
    acc1 = _sc_scatter(n, e, fh)(y1, src, dst, w)
    y2 = _tc_mm2(n, fh, fo, blk)(acc1, y1, dinv, b1.reshape(1, fh), W2)
    acc2 = _sc_scatter(n, e, fo)(y2, src, dst, w)
    out = _tc_fin(n, fo, blk)(acc2, y2, dinv, b2.reshape(1, fo))
    return out


# speedup vs baseline: 18.8984x; 1.1747x over previous
import functools

import jax
import jax.numpy as jnp
from jax import lax
from jax.experimental import pallas as pl
from jax.experimental.pallas import tpu as pltpu
from jax.experimental.pallas import tpu_sc as plsc

_NC = 2
_NS = 16
_K = 80


def _sc_degree(n, e):
    nw = _NC * _NS
    ew = e // nw
    assert e % nw == 0 and ew % _K == 0 and n % _NS == 0
    nchunks = ew // _K
    rpt = (n // 8 // _NS) * 8
    tail = n - rpt * _NS
    nz, rem = rpt // _K, rpt % _K
    mesh = plsc.VectorSubcoreMesh(core_axis_name="c", subcore_axis_name="s")

    assert nchunks >= 3 and nchunks % 2 == 1

    @functools.partial(
        pl.kernel,
        out_type=jax.ShapeDtypeStruct((_NC, n, 16), jnp.float32),
        mesh=mesh,
        compiler_params=pltpu.CompilerParams(use_tc_tiling_on_sc=False),
        scratch_types=[
            pltpu.VMEM((_K,), jnp.int32),
            pltpu.VMEM((_K,), jnp.int32),
            pltpu.VMEM((_K,), jnp.float32),
            pltpu.VMEM((_K,), jnp.float32),
            pltpu.VMEM((_K,), jnp.int32),
            pltpu.VMEM((_K,), jnp.int32),
            pltpu.VMEM((_K, 16), jnp.float32),
            pltpu.VMEM((_K, 16), jnp.float32),
            pltpu.VMEM_SHARED((n, 16), jnp.float32),
            pltpu.SemaphoreType.DMA,
            pltpu.SemaphoreType.DMA,
            pltpu.SemaphoreType.DMA,
            pltpu.SemaphoreType.DMA,
        ],
    )
    def kern(dst_hbm, w_hbm, out_hbm, dstv0, dstv1, wv0, wv1, dsc0, dsc1,
             rows0, rows1, deg_sh, isem0, isem1, ssem0, ssem1):
        c = lax.axis_index("c")
        s = lax.axis_index("s")
        wid = c * _NS + s
        zvec = jnp.zeros((16,), jnp.float32)
        bufs = [
            (dstv0, wv0, dsc0, rows0, isem0, ssem0),
            (dstv1, wv1, dsc1, rows1, isem1, ssem1),
        ]

        @pl.loop(0, _K)
        def _zero_rows(i):
            rows0[i, :] = zvec

        row0 = s * rpt
        for k in range(nz):
            pltpu.sync_copy(rows0.at[pl.ds(0, _K)], deg_sh.at[pl.ds(row0 + k * _K, _K)])
        if rem:
            pltpu.sync_copy(rows0.at[pl.ds(0, rem)], deg_sh.at[pl.ds(row0 + nz * _K, rem)])
        if tail:
            @pl.when(s == _NS - 1)
            def _zero_tail():
                pltpu.sync_copy(rows0.at[pl.ds(0, tail)], deg_sh.at[pl.ds(n - tail, tail)])
        plsc.subcore_barrier()

        def ibase(j):
            return wid * ew + j * _K

        def issue_idx(j, p):
            dstv, wv, _, _, isem, _ = bufs[p]
            b = ibase(j)
            pltpu.async_copy(dst_hbm.at[pl.ds(b, _K)], dstv, isem)
            pltpu.async_copy(w_hbm.at[pl.ds(b, _K)], wv, isem)

        def wait_idx(j, p):
            dstv, wv, _, _, isem, _ = bufs[p]
            b = ibase(j)
            pltpu.make_async_copy(dst_hbm.at[pl.ds(b, _K)], dstv, isem).wait()
            pltpu.make_async_copy(w_hbm.at[pl.ds(b, _K)], wv, isem).wait()

        def issue_scat(p):
            _, _, dsc, rows, _, ssem = bufs[p]
            pltpu.async_copy(rows, deg_sh.at[dsc], ssem, add=True)

        def drain_scat(p):
            _, _, dsc, rows, _, ssem = bufs[p]
            pltpu.make_async_copy(rows, deg_sh.at[dsc], ssem).wait()

        def half(j, p, last):
            dstv, wv, dsc, rows, isem, ssem = bufs[p]
            wait_idx(j, p)
            if last:
                drain_scat(p)
            else:
                @pl.when(j >= 2)
                def _drain_prev2():
                    drain_scat(p)
            for g in range(_K // 16):
                dsc[pl.ds(g * 16, 16)] = dstv[pl.ds(g * 16, 16)]

            @pl.loop(0, _K // 16)
            def _fill(g):
                wvec = wv[pl.ds(g * 16, 16)]
                for l in range(16):
                    rows[g * 16 + l, :] = jnp.full((16,), wvec[l], jnp.float32)

            if not last:
                @pl.when(j + 2 < nchunks)
                def _pf2():
                    issue_idx(j + 2, p)
            issue_scat(p)

        issue_idx(0, 0)
        issue_idx(1, 1)

        @pl.loop(0, nchunks // 2)
        def _pair(t):
            half(2 * t, 0, False)
            half(2 * t + 1, 1, False)

        half(nchunks - 1, 0, True)
        drain_scat(1)
        drain_scat(0)

        plsc.subcore_barrier()
        pltpu.sync_copy(deg_sh.at[pl.ds(row0, rpt)], out_hbm.at[c].at[pl.ds(row0, rpt)])
        if tail:
            @pl.when(s == _NS - 1)
            def _wb_tail():
                pltpu.sync_copy(deg_sh.at[pl.ds(n - tail, tail)],
                                out_hbm.at[c].at[pl.ds(n - tail, tail)])

    return kern


def _sc_scatter(n, e, f):
    nw = _NC * _NS
    ew = e // nw
    assert e % nw == 0 and ew % _K == 0 and n % _NS == 0 and f % 16 == 0
    nchunks = ew // _K
    assert nchunks >= 3 and nchunks % 2 == 1
    rpt = (n // 8 // _NS) * 8
    tail = n - rpt * _NS
    nz, rem = rpt // _K, rpt % _K
    mesh = plsc.VectorSubcoreMesh(core_axis_name="c", subcore_axis_name="s")

    @functools.partial(
        pl.kernel,
        out_type=jax.ShapeDtypeStruct((_NC, n, f), jnp.float32),
        mesh=mesh,
        compiler_params=pltpu.CompilerParams(use_tc_tiling_on_sc=False),
        scratch_types=[
            pltpu.VMEM((_K,), jnp.int32),
            pltpu.VMEM((_K,), jnp.int32),
            pltpu.VMEM((_K,), jnp.int32),
            pltpu.VMEM((_K,), jnp.int32),
            pltpu.VMEM((_K,), jnp.float32),
            pltpu.VMEM((_K,), jnp.float32),
            pltpu.VMEM((_K,), jnp.int32),
            pltpu.VMEM((_K,), jnp.int32),
            pltpu.VMEM((_K, f), jnp.float32),
            pltpu.VMEM((_K, f), jnp.float32),
            pltpu.VMEM_SHARED((n, f), jnp.float32),
            pltpu.SemaphoreType.DMA,
            pltpu.SemaphoreType.DMA,
            pltpu.SemaphoreType.DMA,
            pltpu.SemaphoreType.DMA,
            pltpu.SemaphoreType.DMA,
            pltpu.SemaphoreType.DMA,
        ],
    )
    def kern(y_hbm, src_hbm, dst_hbm, w_hbm, out_hbm,
             srcv0, srcv1, dstv0, dstv1, wv0, wv1, dsc0, dsc1, rows0, rows1,
             acc, isem0, isem1, gsem0, gsem1, ssem0, ssem1):
        c = lax.axis_index("c")
        s = lax.axis_index("s")
        wid = c * _NS + s
        zvec = jnp.zeros((16,), jnp.float32)
        bufs = [
            (srcv0, dstv0, wv0, dsc0, rows0, isem0, gsem0, ssem0),
            (srcv1, dstv1, wv1, dsc1, rows1, isem1, gsem1, ssem1),
        ]

        @pl.loop(0, _K)
        def _zero_rows(i):
            for g in range(f // 16):
                rows0[i, pl.ds(g * 16, 16)] = zvec

        row0 = s * rpt
        for k in range(nz):
            pltpu.sync_copy(rows0.at[pl.ds(0, _K)], acc.at[pl.ds(row0 + k * _K, _K)])
        if rem:
            pltpu.sync_copy(rows0.at[pl.ds(0, rem)], acc.at[pl.ds(row0 + nz * _K, rem)])
        if tail:
            @pl.when(s == _NS - 1)
            def _zero_tail():
                pltpu.sync_copy(rows0.at[pl.ds(0, tail)], acc.at[pl.ds(n - tail, tail)])
        plsc.subcore_barrier()

        def ibase(j):
            return wid * ew + j * _K

        def issue_idx(j, p):
            srcv, dstv, wv, _, _, isem, _, _ = bufs[p]
            b = ibase(j)
            pltpu.async_copy(src_hbm.at[pl.ds(b, _K)], srcv, isem)
            pltpu.async_copy(dst_hbm.at[pl.ds(b, _K)], dstv, isem)
            pltpu.async_copy(w_hbm.at[pl.ds(b, _K)], wv, isem)

        def wait_idx(j, p):
            srcv, dstv, wv, _, _, isem, _, _ = bufs[p]
            b = ibase(j)
            pltpu.make_async_copy(src_hbm.at[pl.ds(b, _K)], srcv, isem).wait()
            pltpu.make_async_copy(dst_hbm.at[pl.ds(b, _K)], dstv, isem).wait()
            pltpu.make_async_copy(w_hbm.at[pl.ds(b, _K)], wv, isem).wait()

        def issue_gather(p):
            srcv, _, _, _, rows, _, gsem, _ = bufs[p]
            pltpu.async_copy(y_hbm.at[srcv], rows, gsem)

        def wait_gather(p):
            srcv, _, _, _, rows, _, gsem, _ = bufs[p]
            pltpu.make_async_copy(y_hbm.at[srcv], rows, gsem).wait()

        def issue_scat(p):
            _, _, _, dsc, rows, _, _, ssem = bufs[p]
            pltpu.async_copy(rows, acc.at[dsc], ssem, add=True)

        def drain_scat(p):
            _, _, _, dsc, rows, _, _, ssem = bufs[p]
            pltpu.make_async_copy(rows, acc.at[dsc], ssem).wait()

        def half(j, p, last):
            srcv, dstv, wv, dsc, rows, isem, gsem, ssem = bufs[p]
            pn = 1 - p
            wait_gather(p)
            if last:
                drain_scat(pn)
            else:
                @pl.when(j >= 1)
                def _drain_prev():
                    drain_scat(pn)

                wait_idx(j + 1, pn)
                issue_gather(pn)
            for g in range(_K // 16):
                dsc[pl.ds(g * 16, 16)] = dstv[pl.ds(g * 16, 16)]

            @pl.loop(0, _K // 16)
            def _scale(g):
                wvec = wv[pl.ds(g * 16, 16)]
                for l in range(16):
                    wi = wvec[l]
                    i = g * 16 + l
                    for fb in range(f // 16):
                        rows[i, pl.ds(fb * 16, 16)] = rows[i, pl.ds(fb * 16, 16)] * wi

            if not last:
                @pl.when(j + 2 < nchunks)
                def _pf2():
                    issue_idx(j + 2, p)
            issue_scat(p)

        issue_idx(0, 0)
        issue_idx(1, 1)
        wait_idx(0, 0)
        issue_gather(0)

        @pl.loop(0, nchunks // 2)
        def _pair(t):
            half(2 * t, 0, False)
            half(2 * t + 1, 1, False)

        half(nchunks - 1, 0, True)
        drain_scat(0)

        plsc.subcore_barrier()
        pltpu.sync_copy(acc.at[pl.ds(row0, rpt)], out_hbm.at[c].at[pl.ds(row0, rpt)])
        if tail:
            @pl.when(s == _NS - 1)
            def _wb_tail():
                pltpu.sync_copy(acc.at[pl.ds(n - tail, tail)],
                                out_hbm.at[c].at[pl.ds(n - tail, tail)])

    return kern


def _tc_mm1(n, fin, fh, blk):
    def body(deg_ref, x_ref, w_ref, y_ref, dinv_ref):
        d = deg_ref[0, :, 0:1] + deg_ref[1, :, 0:1] + 1.0
        dinv = jnp.where(d > 0, lax.rsqrt(d), 0.0)
        xw = jnp.dot(x_ref[...], w_ref[...], preferred_element_type=jnp.float32,
                     precision=lax.Precision.HIGHEST)
        y_ref[...] = dinv * xw
        dinv_ref[...] = dinv

    return pl.pallas_call(
        body,
        grid=(n // blk,),
        in_specs=[
            pl.BlockSpec((_NC, blk, 16), lambda i: (0, i, 0)),
            pl.BlockSpec((blk, fin), lambda i: (i, 0)),
            pl.BlockSpec((fin, fh), lambda i: (0, 0)),
        ],
        out_specs=[
            pl.BlockSpec((blk, fh), lambda i: (i, 0)),
            pl.BlockSpec((blk, 1), lambda i: (i, 0)),
        ],
        out_shape=[
            jax.ShapeDtypeStruct((n, fh), jnp.float32),
            jax.ShapeDtypeStruct((n, 1), jnp.float32),
        ],
    )


def _tc_mm2(n, fh, fo, blk):
    def body(acc_ref, y1_ref, dinv_ref, b1_ref, w2_ref, y2_ref):
        dinv = dinv_ref[...]
        a = acc_ref[0] + acc_ref[1] + y1_ref[...]
        h = jnp.maximum(dinv * a + b1_ref[...], 0.0)
        y2_ref[...] = dinv * jnp.dot(h, w2_ref[...], preferred_element_type=jnp.float32,
                                     precision=lax.Precision.HIGHEST)

    return pl.pallas_call(
        body,
        grid=(n // blk,),
        in_specs=[
            pl.BlockSpec((_NC, blk, fh), lambda i: (0, i, 0)),
            pl.BlockSpec((blk, fh), lambda i: (i, 0)),
            pl.BlockSpec((blk, 1), lambda i: (i, 0)),
            pl.BlockSpec((1, fh), lambda i: (0, 0)),
            pl.BlockSpec((fh, fo), lambda i: (0, 0)),
        ],
        out_specs=pl.BlockSpec((blk, fo), lambda i: (i, 0)),
        out_shape=jax.ShapeDtypeStruct((n, fo), jnp.float32),
    )


def _tc_fin(n, fo, blk):
    def body(acc_ref, y2_ref, dinv_ref, b2_ref, out_ref):
        out_ref[...] = dinv_ref[...] * (acc_ref[0] + acc_ref[1] + y2_ref[...]) + b2_ref[...]

    return pl.pallas_call(
        body,
        grid=(n // blk,),
        in_specs=[
            pl.BlockSpec((_NC, blk, fo), lambda i: (0, i, 0)),
            pl.BlockSpec((blk, fo), lambda i: (i, 0)),
            pl.BlockSpec((blk, 1), lambda i: (i, 0)),
            pl.BlockSpec((1, fo), lambda i: (0, 0)),
        ],
        out_specs=pl.BlockSpec((blk, fo), lambda i: (i, 0)),
        out_shape=jax.ShapeDtypeStruct((n, fo), jnp.float32),
    )


def kernel(in_feat, edge_index, edge_weight, W1, b1, W2, b2):
    n, fin = in_feat.shape
    e = edge_index.shape[1]
    fh = W1.shape[1]
    fo = W2.shape[1]
    blk = 1000
    assert n % blk == 0

    src = edge_index[0].astype(jnp.int32)
    dst = edge_index[1].astype(jnp.int32)
    w = edge_weight.astype(jnp.float32)

    deg2 = _sc_degree(n, e)(dst, w)
    y1, dinv = _tc_mm1(n, fin, fh, blk)(deg2, in_feat.astype(jnp.float32), W1)
    acc1 = _sc_scatter(n, e, fh)(y1, src, dst, w)
    y2 = _tc_mm2(n, fh, fo, blk)(acc1, y1, dinv, b1.reshape(1, fh), W2)
    acc2 = _sc_scatter(n, e, fo)(y2, src, dst, w)
    out = _tc_fin(n, fo, blk)(acc2, y2, dinv, b2.reshape(1, fo))
    return out


# speedup vs baseline: 22.7552x; 1.2041x over previous
import functools

import jax
import jax.numpy as jnp
from jax import lax
from jax.experimental import pallas as pl
from jax.experimental.pallas import tpu as pltpu
from jax.experimental.pallas import tpu_sc as plsc

_NC = 2
_NS = 16
_K = 80


def _sc_degree(n, e):
    nw = _NC * _NS
    ew = e // nw
    assert e % nw == 0 and ew % _K == 0 and n % _NS == 0
    nchunks = ew // _K
    rpt = (n // 8 // _NS) * 8
    tail = n - rpt * _NS
    nz, rem = rpt // _K, rpt % _K
    mesh = plsc.VectorSubcoreMesh(core_axis_name="c", subcore_axis_name="s")

    assert nchunks >= 3 and nchunks % 2 == 1

    @functools.partial(
        pl.kernel,
        out_type=jax.ShapeDtypeStruct((_NC, n, 16), jnp.float32),
        mesh=mesh,
        compiler_params=pltpu.CompilerParams(use_tc_tiling_on_sc=False),
        scratch_types=[
            pltpu.VMEM((_K,), jnp.int32),
            pltpu.VMEM((_K,), jnp.int32),
            pltpu.VMEM((_K,), jnp.float32),
            pltpu.VMEM((_K,), jnp.float32),
            pltpu.VMEM((_K,), jnp.int32),
            pltpu.VMEM((_K,), jnp.int32),
            pltpu.VMEM((_K, 16), jnp.float32),
            pltpu.VMEM((_K, 16), jnp.float32),
            pltpu.VMEM_SHARED((n, 16), jnp.float32),
            pltpu.SemaphoreType.DMA,
            pltpu.SemaphoreType.DMA,
            pltpu.SemaphoreType.DMA,
            pltpu.SemaphoreType.DMA,
        ],
    )
    def kern(dst_hbm, w_hbm, out_hbm, dstv0, dstv1, wv0, wv1, dsc0, dsc1,
             rows0, rows1, deg_sh, isem0, isem1, ssem0, ssem1):
        c = lax.axis_index("c")
        s = lax.axis_index("s")
        wid = c * _NS + s
        zvec = jnp.zeros((16,), jnp.float32)
        bufs = [
            (dstv0, wv0, dsc0, rows0, isem0, ssem0),
            (dstv1, wv1, dsc1, rows1, isem1, ssem1),
        ]

        @pl.loop(0, _K)
        def _zero_rows(i):
            rows0[i, :] = zvec

        row0 = s * rpt
        for k in range(nz):
            pltpu.sync_copy(rows0.at[pl.ds(0, _K)], deg_sh.at[pl.ds(row0 + k * _K, _K)])
        if rem:
            pltpu.sync_copy(rows0.at[pl.ds(0, rem)], deg_sh.at[pl.ds(row0 + nz * _K, rem)])
        if tail:
            @pl.when(s == _NS - 1)
            def _zero_tail():
                pltpu.sync_copy(rows0.at[pl.ds(0, tail)], deg_sh.at[pl.ds(n - tail, tail)])
        plsc.subcore_barrier()

        def ibase(j):
            return wid * ew + j * _K

        def issue_idx(j, p):
            dstv, wv, _, _, isem, _ = bufs[p]
            b = ibase(j)
            pltpu.async_copy(dst_hbm.at[pl.ds(b, _K)], dstv, isem)
            pltpu.async_copy(w_hbm.at[pl.ds(b, _K)], wv, isem)

        def wait_idx(j, p):
            dstv, wv, _, _, isem, _ = bufs[p]
            b = ibase(j)
            pltpu.make_async_copy(dst_hbm.at[pl.ds(b, _K)], dstv, isem).wait()
            pltpu.make_async_copy(w_hbm.at[pl.ds(b, _K)], wv, isem).wait()

        def issue_scat(p):
            _, _, dsc, rows, _, ssem = bufs[p]
            pltpu.async_copy(rows, deg_sh.at[dsc], ssem, add=True)

        def drain_scat(p):
            _, _, dsc, rows, _, ssem = bufs[p]
            pltpu.make_async_copy(rows, deg_sh.at[dsc], ssem).wait()

        def half(j, p, last):
            dstv, wv, dsc, rows, isem, ssem = bufs[p]
            wait_idx(j, p)
            if last:
                drain_scat(p)
            else:
                @pl.when(j >= 2)
                def _drain_prev2():
                    drain_scat(p)
            for g in range(_K // 16):
                dsc[pl.ds(g * 16, 16)] = dstv[pl.ds(g * 16, 16)]

            @pl.loop(0, _K // 16)
            def _fill(g):
                wvec = wv[pl.ds(g * 16, 16)]
                for l in range(16):
                    rows[g * 16 + l, :] = jnp.full((16,), wvec[l], jnp.float32)

            if not last:
                @pl.when(j + 2 < nchunks)
                def _pf2():
                    issue_idx(j + 2, p)
            issue_scat(p)

        issue_idx(0, 0)
        issue_idx(1, 1)

        @pl.loop(0, nchunks // 2)
        def _pair(t):
            half(2 * t, 0, False)
            half(2 * t + 1, 1, False)

        half(nchunks - 1, 0, True)
        drain_scat(1)
        drain_scat(0)

        plsc.subcore_barrier()
        pltpu.sync_copy(deg_sh.at[pl.ds(row0, rpt)], out_hbm.at[c].at[pl.ds(row0, rpt)])
        if tail:
            @pl.when(s == _NS - 1)
            def _wb_tail():
                pltpu.sync_copy(deg_sh.at[pl.ds(n - tail, tail)],
                                out_hbm.at[c].at[pl.ds(n - tail, tail)])

    return kern


def _sc_scatter(n, e, f):
    nw = _NC * _NS
    ew = e // nw
    assert e % nw == 0 and ew % _K == 0 and n % _NS == 0 and f % 16 == 0
    nchunks = ew // _K
    assert nchunks >= 3 and nchunks % 2 == 1
    rpt = (n // 8 // _NS) * 8
    tail = n - rpt * _NS
    nz, rem = rpt // _K, rpt % _K
    mesh = plsc.VectorSubcoreMesh(core_axis_name="c", subcore_axis_name="s")

    @functools.partial(
        pl.kernel,
        out_type=jax.ShapeDtypeStruct((_NC, n, f), jnp.float32),
        mesh=mesh,
        compiler_params=pltpu.CompilerParams(use_tc_tiling_on_sc=False),
        scratch_types=[
            pltpu.VMEM((_K,), jnp.int32),
            pltpu.VMEM((_K,), jnp.int32),
            pltpu.VMEM((_K,), jnp.int32),
            pltpu.VMEM((_K,), jnp.int32),
            pltpu.VMEM((_K,), jnp.float32),
            pltpu.VMEM((_K,), jnp.float32),
            pltpu.VMEM((_K,), jnp.int32),
            pltpu.VMEM((_K,), jnp.int32),
            pltpu.VMEM((_K, f), jnp.float32),
            pltpu.VMEM((_K, f), jnp.float32),
            pltpu.VMEM_SHARED((n, f), jnp.float32),
            pltpu.SemaphoreType.DMA,
            pltpu.SemaphoreType.DMA,
            pltpu.SemaphoreType.DMA,
            pltpu.SemaphoreType.DMA,
            pltpu.SemaphoreType.DMA,
            pltpu.SemaphoreType.DMA,
        ],
    )
    def kern(y_hbm, src_hbm, dst_hbm, w_hbm, out_hbm,
             srcv0, srcv1, dstv0, dstv1, wv0, wv1, dsc0, dsc1, rows0, rows1,
             acc, isem0, isem1, gsem0, gsem1, ssem0, ssem1):
        c = lax.axis_index("c")
        s = lax.axis_index("s")
        wid = c * _NS + s
        zvec = jnp.zeros((16,), jnp.float32)
        bufs = [
            (srcv0, dstv0, wv0, dsc0, rows0, isem0, gsem0, ssem0),
            (srcv1, dstv1, wv1, dsc1, rows1, isem1, gsem1, ssem1),
        ]

        @pl.loop(0, _K)
        def _zero_rows(i):
            for g in range(f // 16):
                rows0[i, pl.ds(g * 16, 16)] = zvec

        row0 = s * rpt
        for k in range(nz):
            pltpu.sync_copy(rows0.at[pl.ds(0, _K)], acc.at[pl.ds(row0 + k * _K, _K)])
        if rem:
            pltpu.sync_copy(rows0.at[pl.ds(0, rem)], acc.at[pl.ds(row0 + nz * _K, rem)])
        if tail:
            @pl.when(s == _NS - 1)
            def _zero_tail():
                pltpu.sync_copy(rows0.at[pl.ds(0, tail)], acc.at[pl.ds(n - tail, tail)])
        plsc.subcore_barrier()

        def ibase(j):
            return wid * ew + j * _K

        def issue_idx(j, p):
            srcv, dstv, wv, _, _, isem, _, _ = bufs[p]
            b = ibase(j)
            pltpu.async_copy(src_hbm.at[pl.ds(b, _K)], srcv, isem)
            pltpu.async_copy(dst_hbm.at[pl.ds(b, _K)], dstv, isem)
            pltpu.async_copy(w_hbm.at[pl.ds(b, _K)], wv, isem)

        def wait_idx(j, p):
            srcv, dstv, wv, _, _, isem, _, _ = bufs[p]
            b = ibase(j)
            pltpu.make_async_copy(src_hbm.at[pl.ds(b, _K)], srcv, isem).wait()
            pltpu.make_async_copy(dst_hbm.at[pl.ds(b, _K)], dstv, isem).wait()
            pltpu.make_async_copy(w_hbm.at[pl.ds(b, _K)], wv, isem).wait()

        def issue_gather(p):
            srcv, _, _, _, rows, _, gsem, _ = bufs[p]
            pltpu.async_copy(y_hbm.at[srcv], rows, gsem)

        def wait_gather(p):
            srcv, _, _, _, rows, _, gsem, _ = bufs[p]
            pltpu.make_async_copy(y_hbm.at[srcv], rows, gsem).wait()

        def issue_scat(p):
            _, _, _, dsc, rows, _, _, ssem = bufs[p]
            pltpu.async_copy(rows, acc.at[dsc], ssem, add=True)

        def drain_scat(p):
            _, _, _, dsc, rows, _, _, ssem = bufs[p]
            pltpu.make_async_copy(rows, acc.at[dsc], ssem).wait()

        def half(j, p, last):
            srcv, dstv, wv, dsc, rows, isem, gsem, ssem = bufs[p]
            pn = 1 - p
            wait_gather(p)
            if last:
                drain_scat(pn)
            else:
                @pl.when(j >= 1)
                def _drain_prev():
                    drain_scat(pn)

                wait_idx(j + 1, pn)
                issue_gather(pn)
            for g in range(_K // 16):
                dsc[pl.ds(g * 16, 16)] = dstv[pl.ds(g * 16, 16)]

            @pl.loop(0, _K // 16)
            def _scale(g):
                wvec = wv[pl.ds(g * 16, 16)]
                for l in range(16):
                    wi = wvec[l]
                    i = g * 16 + l
                    for fb in range(f // 16):
                        rows[i, pl.ds(fb * 16, 16)] = rows[i, pl.ds(fb * 16, 16)] * wi

            if not last:
                @pl.when(j + 2 < nchunks)
                def _pf2():
                    issue_idx(j + 2, p)
            issue_scat(p)

        issue_idx(0, 0)
        issue_idx(1, 1)
        wait_idx(0, 0)
        issue_gather(0)

        @pl.loop(0, nchunks // 2)
        def _pair(t):
            half(2 * t, 0, False)
            half(2 * t + 1, 1, False)

        half(nchunks - 1, 0, True)
        drain_scat(0)

        plsc.subcore_barrier()
        pltpu.sync_copy(acc.at[pl.ds(row0, rpt)], out_hbm.at[c].at[pl.ds(row0, rpt)])
        if tail:
            @pl.when(s == _NS - 1)
            def _wb_tail():
                pltpu.sync_copy(acc.at[pl.ds(n - tail, tail)],
                                out_hbm.at[c].at[pl.ds(n - tail, tail)])

    return kern


def _tc_mm1(n, fin, fh, blk):
    def body(deg_ref, x_ref, w_ref, y_ref, dinv_ref):
        d = deg_ref[0, :, 0:1] + deg_ref[1, :, 0:1] + 1.0
        dinv = jnp.where(d > 0, lax.rsqrt(d), 0.0)
        xw = jnp.dot(x_ref[...], w_ref[...], preferred_element_type=jnp.float32,
                     precision=lax.Precision.HIGHEST)
        y_ref[...] = dinv * xw
        dinv_ref[...] = dinv

    return pl.pallas_call(
        body,
        grid=(n // blk,),
        in_specs=[
            pl.BlockSpec((_NC, blk, 16), lambda i: (0, i, 0)),
            pl.BlockSpec((blk, fin), lambda i: (i, 0)),
            pl.BlockSpec((fin, fh), lambda i: (0, 0)),
        ],
        out_specs=[
            pl.BlockSpec((blk, fh), lambda i: (i, 0)),
            pl.BlockSpec((blk, 1), lambda i: (i, 0)),
        ],
        out_shape=[
            jax.ShapeDtypeStruct((n, fh), jnp.float32),
            jax.ShapeDtypeStruct((n, 1), jnp.float32),
        ],
    )


def _tc_mm2(n, fh, fp, blk):
    def body(acc_ref, y1_ref, dinv_ref, b1_ref, w2_ref, y2_ref):
        dinv = dinv_ref[...]
        a = acc_ref[0] + acc_ref[1] + y1_ref[...]
        h = jnp.maximum(dinv * a + b1_ref[...], 0.0)
        y2_ref[...] = dinv * jnp.dot(h, w2_ref[...], preferred_element_type=jnp.float32,
                                     precision=lax.Precision.HIGHEST)

    return pl.pallas_call(
        body,
        grid=(n // blk,),
        in_specs=[
            pl.BlockSpec((_NC, blk, fh), lambda i: (0, i, 0)),
            pl.BlockSpec((blk, fh), lambda i: (i, 0)),
            pl.BlockSpec((blk, 1), lambda i: (i, 0)),
            pl.BlockSpec((1, fh), lambda i: (0, 0)),
            pl.BlockSpec((fh, fp), lambda i: (0, 0)),
        ],
        out_specs=pl.BlockSpec((blk, fp), lambda i: (i, 0)),
        out_shape=jax.ShapeDtypeStruct((n, fp), jnp.float32),
    )


def _tc_fin(n, fo, fp, blk):
    def body(acc_ref, y2_ref, dinv_ref, b2_ref, out_ref):
        a = (acc_ref[0] + acc_ref[1] + y2_ref[...])[:, :fo]
        out_ref[...] = dinv_ref[...] * a + b2_ref[...]

    return pl.pallas_call(
        body,
        grid=(n // blk,),
        in_specs=[
            pl.BlockSpec((_NC, blk, fp), lambda i: (0, i, 0)),
            pl.BlockSpec((blk, fp), lambda i: (i, 0)),
            pl.BlockSpec((blk, 1), lambda i: (i, 0)),
            pl.BlockSpec((1, fo), lambda i: (0, 0)),
        ],
        out_specs=pl.BlockSpec((blk, fo), lambda i: (i, 0)),
        out_shape=jax.ShapeDtypeStruct((n, fo), jnp.float32),
    )


def kernel(in_feat, edge_index, edge_weight, W1, b1, W2, b2):
    n, fin = in_feat.shape
    e = edge_index.shape[1]
    fh = W1.shape[1]
    fo = W2.shape[1]
    blk = 1000
    assert n % blk == 0

    src = edge_index[0].astype(jnp.int32)
    dst = edge_index[1].astype(jnp.int32)
    w = edge_weight.astype(jnp.float32)

    fp = max(fo, 128)
    W2p = jnp.concatenate([W2, jnp.zeros((fh, fp - fo), jnp.float32)], axis=1) if fp > fo else W2

    deg2 = _sc_degree(n, e)(dst, w)
    y1, dinv = _tc_mm1(n, fin, fh, blk)(deg2, in_feat.astype(jnp.float32), W1)
    acc1 = _sc_scatter(n, e, fh)(y1, src, dst, w)
    y2 = _tc_mm2(n, fh, fp, blk)(acc1, y1, dinv, b1.reshape(1, fh), W2p)
    acc2 = _sc_scatter(n, e, fp)(y2, src, dst, w)
    out = _tc_fin(n, fo, fp, blk)(acc2, y2, dinv, b2.reshape(1, fo))
    return out
